# baseline (speedup 1.0000x reference)
import numpy as np
import jax
import jax.numpy as jnp
from jax import lax
from jax.experimental import pallas as pl
from jax.experimental.pallas import tpu as pltpu
from jax.experimental.pallas import tpu_sc as plsc

N = 50000
E = 800000
H = 64
FLT = 64
G = 50
CUT = 10.0
NB = 128
LAYERS = 6
HH = H // 2

NC = 2
NS = 16
E_PAD = 802816
CHUNK_A = E_PAD // (NC * NS)
GROUPS_A = CHUNK_A // 16
CHUNK_D = E_PAD // NS
EB = 128
NBLK_D = CHUNK_D // EB
AGG_ROWS = 50048
RPT = AGG_ROWS // NS
ZROWS = RPT // 8
BN = 2000
NGRID = N // BN
EP4 = E_PAD // 4
BEP = 1024
EGRID = EP4 // BEP
EB4 = EB // 4

_OFF_NP = np.linspace(0.0, CUT, G).astype(np.float32)
_COEFF = float(np.float32(-0.5) / (_OFF_NP[1] - _OFF_NP[0]) ** 2)
_LN2 = float(np.float32(np.log(2.0)))
_PI = float(np.float32(np.pi))


def _ssp(x):
    return jnp.maximum(x, 0.0) + jnp.log(1.0 + jnp.exp(-jnp.abs(x))) - _LN2


def _geom_body(posx, posy, posz, rowh, colh, d2h, ridx, cidx, acc, ptab):
    cid = lax.axis_index("c")
    sid = lax.axis_index("s")
    wid = sid * NC + cid
    base = wid * CHUNK_A
    pltpu.sync_copy(rowh.at[pl.ds(base, CHUNK_A)], ridx)
    pltpu.sync_copy(colh.at[pl.ds(base, CHUNK_A)], cidx)
    for ci, ph in enumerate((posx, posy, posz)):
        pltpu.sync_copy(ph, ptab)

        def body(g, carry, first=(ci == 0)):
            for u in range(8):
                off = g * 128 + u * 16
                rv = ridx[pl.ds(off, 16)]
                cv = cidx[pl.ds(off, 16)]
                a = plsc.load_gather(ptab, [rv])
                b = plsc.load_gather(ptab, [cv])
                d = a - b
                sq = d * d
                if first:
                    acc[pl.ds(off, 16)] = sq
                else:
                    acc[pl.ds(off, 16)] = acc[pl.ds(off, 16)] + sq
            return carry

        lax.fori_loop(0, GROUPS_A // 8, body, 0)
    pltpu.sync_copy(acc, d2h.at[pl.ds(base, CHUNK_A)])


def _geom(*args):
    return pl.kernel(
        _geom_body,
        out_type=jax.ShapeDtypeStruct((E_PAD,), jnp.float32),
        mesh=plsc.VectorSubcoreMesh(core_axis_name="c", subcore_axis_name="s",
                                    num_cores=NC, num_subcores=NS),
        compiler_params=pltpu.CompilerParams(needs_layout_passes=False,
                                             use_tc_tiling_on_sc=False),
        scratch_types=[
            pltpu.VMEM((CHUNK_A,), jnp.int32),
            pltpu.VMEM((CHUNK_A,), jnp.int32),
            pltpu.VMEM((CHUNK_A,), jnp.float32),
            pltpu.VMEM((N,), jnp.float32),
        ],
    )(*args)


def _scat_body(vlincat, wfp, jcat, icat, outcat,
               jbuf, ibuf, vrows, wfbuf, zbuf, agg, sem):
    cid = lax.axis_index("c")
    sid = lax.axis_index("s")
    tbase = sid * CHUNK_D

    def zb(r, carry):
        z16 = jnp.zeros((16,), jnp.float32)
        zbuf[r, pl.ds(0, 16)] = z16
        zbuf[r, pl.ds(16, 16)] = z16
        return carry

    lax.fori_loop(0, ZROWS, zb, 0)
    for k in range(8):
        pltpu.sync_copy(zbuf, agg.at[pl.ds(sid * RPT + k * ZROWS, ZROWS)])
    plsc.subcore_barrier()

    joff = cid * E_PAD
    tbase4 = sid * (CHUNK_D // 4)

    def blk(b, carry):
        eb = tbase + b * EB
        pltpu.sync_copy(jcat.at[pl.ds(joff + eb, EB)], jbuf)
        pltpu.sync_copy(icat.at[pl.ds(eb, EB)], ibuf)
        gcp = pltpu.async_copy(vlincat.at[jbuf], vrows, sem)
        pltpu.sync_copy(wfp.at[cid].at[pl.ds(tbase4 + b * EB4, EB4)], wfbuf)
        gcp.wait()

        def mul(r4, c2):
            for j in range(4):
                for k in range(2):
                    vrows[r4 * 4 + j, pl.ds(k * 16, 16)] = (
                        vrows[r4 * 4 + j, pl.ds(k * 16, 16)]
                        * wfbuf[r4, pl.ds(j * 32 + k * 16, 16)])
            return c2

        lax.fori_loop(0, EB4, mul, 0)
        pltpu.sync_copy(vrows, agg.at[ibuf], add=True)
        return carry

    lax.fori_loop(0, NBLK_D, blk, 0)
    plsc.subcore_barrier()
    pltpu.sync_copy(agg.at[pl.ds(sid * RPT, RPT)],
                    outcat.at[pl.ds(cid * AGG_ROWS + sid * RPT, RPT)])


def _scat(*args):
    return pl.kernel(
        _scat_body,
        out_type=jax.ShapeDtypeStruct((NC * AGG_ROWS, HH), jnp.float32),
        mesh=plsc.VectorSubcoreMesh(core_axis_name="c", subcore_axis_name="s",
                                    num_cores=NC, num_subcores=NS),
        compiler_params=pltpu.CompilerParams(needs_layout_passes=False,
                                             use_tc_tiling_on_sc=False),
        scratch_types=[
            pltpu.VMEM((EB,), jnp.int32),
            pltpu.VMEM((EB,), jnp.int32),
            pltpu.VMEM((EB, HH), jnp.float32),
            pltpu.VMEM((EB4, 128), jnp.float32),
            pltpu.VMEM((ZROWS, HH), jnp.float32),
            pltpu.VMEM_SHARED((AGG_ROWS, HH), jnp.float32),
            pltpu.SemaphoreType.DMA,
        ],
    )(*args)


def _k0_body(z_ref, emb_ref, mL_ref, mR_ref, v_ref, vlin_ref):
    zc = z_ref[...]
    oh = (zc == lax.broadcasted_iota(jnp.int32, (BN, 100), 1)).astype(jnp.float32)
    v = jnp.dot(oh, emb_ref[...], preferred_element_type=jnp.float32,
                   precision=lax.Precision.HIGHEST)
    v_ref[...] = v
    vlin_ref[0] = jnp.dot(v, mL_ref[...], preferred_element_type=jnp.float32,
                   precision=lax.Precision.HIGHEST)
    vlin_ref[1] = jnp.dot(v, mR_ref[...], preferred_element_type=jnp.float32,
                   precision=lax.Precision.HIGHEST)


def _k0(zcol, emb, mL, mR):
    return pl.pallas_call(
        _k0_body,
        grid=(NGRID,),
        in_specs=[
            pl.BlockSpec((BN, 1), lambda b: (b, 0)),
            pl.BlockSpec((100, H), lambda b: (0, 0)),
            pl.BlockSpec((H, HH), lambda b: (0, 0)),
            pl.BlockSpec((H, HH), lambda b: (0, 0)),
        ],
        out_specs=[
            pl.BlockSpec((BN, H), lambda b: (b, 0)),
            pl.BlockSpec((2, BN, HH), lambda b: (0, b, 0)),
        ],
        out_shape=[
            jax.ShapeDtypeStruct((N, H), jnp.float32),
            jax.ShapeDtypeStruct((2, N, HH), jnp.float32),
        ],
    )(zcol, emb, mL, mR)


def _smear_body(d2_ref, offw_ref, e4_ref, e4b_ref, demb_ref, cc_ref):
    distp = jnp.sqrt(d2_ref[...] + 1e-12)
    dw = jnp.dot(distp, e4_ref[...], preferred_element_type=jnp.float32,
                 precision=lax.Precision.HIGHEST)
    dd = dw - offw_ref[...]
    demb_ref[...] = jnp.exp(_COEFF * (dd * dd))
    dwb = jnp.dot(distp, e4b_ref[...], preferred_element_type=jnp.float32,
                  precision=lax.Precision.HIGHEST)
    cc_ref[...] = 0.5 * (jnp.cos(dwb * _PI / CUT) + 1.0)


def _smear(d2p, offw, e4, e4b):
    return pl.pallas_call(
        _smear_body,
        grid=(EGRID,),
        in_specs=[
            pl.BlockSpec((BEP, 4), lambda e: (e, 0)),
            pl.BlockSpec((1, 4 * G), lambda e: (0, 0)),
            pl.BlockSpec((4, 4 * G), lambda e: (0, 0)),
            pl.BlockSpec((4, 128), lambda e: (0, 0)),
        ],
        out_specs=[
            pl.BlockSpec((BEP, 4 * G), lambda e: (e, 0)),
            pl.BlockSpec((BEP, 128), lambda e: (e, 0)),
        ],
        out_shape=[
            jax.ShapeDtypeStruct((EP4, 4 * G), jnp.float32),
            jax.ShapeDtypeStruct((EP4, 128), jnp.float32),
        ],
    )(d2p, offw, e4, e4b)


def _wf_body(demb_ref, cc_ref, w1_ref, b1_ref, w2L_ref, b2L_ref,
             w2R_ref, b2R_ref, out_ref):
    cc = cc_ref[...]
    h1 = _ssp(jnp.dot(demb_ref[...], w1_ref[...],
                      preferred_element_type=jnp.float32,
                      precision=lax.Precision.HIGHEST)
              + b1_ref[...])
    out_ref[0] = (jnp.dot(h1, w2L_ref[...], preferred_element_type=jnp.float32,
                          precision=lax.Precision.HIGHEST)
                  + b2L_ref[...]) * cc
    out_ref[1] = (jnp.dot(h1, w2R_ref[...], preferred_element_type=jnp.float32,
                          precision=lax.Precision.HIGHEST)
                  + b2R_ref[...]) * cc


def _wf(demb, ccp, w1blk, b1t, w2L, b2L, w2R, b2R):
    return pl.pallas_call(
        _wf_body,
        grid=(EGRID,),
        in_specs=[
            pl.BlockSpec((BEP, 4 * G), lambda e: (e, 0)),
            pl.BlockSpec((BEP, 128), lambda e: (e, 0)),
            pl.BlockSpec((4 * G, 4 * FLT), lambda e: (0, 0)),
            pl.BlockSpec((1, 4 * FLT), lambda e: (0, 0)),
            pl.BlockSpec((4 * FLT, 128), lambda e: (0, 0)),
            pl.BlockSpec((1, 128), lambda e: (0, 0)),
            pl.BlockSpec((4 * FLT, 128), lambda e: (0, 0)),
            pl.BlockSpec((1, 128), lambda e: (0, 0)),
        ],
        out_specs=pl.BlockSpec((2, BEP, 128), lambda e: (0, e, 0)),
        out_shape=jax.ShapeDtypeStruct((2, EP4, 128), jnp.float32),
    )(demb, ccp, w1blk, b1t, w2L, b2L, w2R, b2R)


def _mlp_body(aL_ref, aR_ref, v_ref, w1L_ref, w1R_ref, b1_ref, w2_ref,
              b2_ref, mL_ref, mR_ref, vn_ref, vlin_ref):
    a = (jnp.dot(aL_ref[...], w1L_ref[...], preferred_element_type=jnp.float32,
                   precision=lax.Precision.HIGHEST)
         + jnp.dot(aR_ref[...], w1R_ref[...], preferred_element_type=jnp.float32,
                   precision=lax.Precision.HIGHEST)
         + b1_ref[...])
    h = _ssp(a)
    o = jnp.dot(h, w2_ref[...], preferred_element_type=jnp.float32,
                   precision=lax.Precision.HIGHEST) + b2_ref[...]
    vn = v_ref[...] + o
    vn_ref[...] = vn
    vlin_ref[0] = jnp.dot(vn, mL_ref[...], preferred_element_type=jnp.float32,
                   precision=lax.Precision.HIGHEST)
    vlin_ref[1] = jnp.dot(vn, mR_ref[...], preferred_element_type=jnp.float32,
                   precision=lax.Precision.HIGHEST)


def _mlp_last_body(aL_ref, aR_ref, v_ref, w1L_ref, w1R_ref, b1_ref, w2_ref,
                   b2_ref, vn_ref):
    a = (jnp.dot(aL_ref[...], w1L_ref[...], preferred_element_type=jnp.float32,
                   precision=lax.Precision.HIGHEST)
         + jnp.dot(aR_ref[...], w1R_ref[...], preferred_element_type=jnp.float32,
                   precision=lax.Precision.HIGHEST)
         + b1_ref[...])
    h = _ssp(a)
    o = jnp.dot(h, w2_ref[...], preferred_element_type=jnp.float32,
                   precision=lax.Precision.HIGHEST) + b2_ref[...]
    vn_ref[...] = v_ref[...] + o


_MLP_IN_SPECS = [
    pl.BlockSpec((BN, HH), lambda b: (b, 0)),
    pl.BlockSpec((BN, HH), lambda b: (b, 0)),
    pl.BlockSpec((BN, H), lambda b: (b, 0)),
    pl.BlockSpec((HH, H), lambda b: (0, 0)),
    pl.BlockSpec((HH, H), lambda b: (0, 0)),
    pl.BlockSpec((1, H), lambda b: (0, 0)),
    pl.BlockSpec((H, H), lambda b: (0, 0)),
    pl.BlockSpec((1, H), lambda b: (0, 0)),
]


def _mlp(aL, aR, v, w1L, w1R, b1, w2, b2, mL, mR):
    return pl.pallas_call(
        _mlp_body,
        grid=(NGRID,),
        in_specs=_MLP_IN_SPECS + [
            pl.BlockSpec((H, HH), lambda b: (0, 0)),
            pl.BlockSpec((H, HH), lambda b: (0, 0)),
        ],
        out_specs=[
            pl.BlockSpec((BN, H), lambda b: (b, 0)),
            pl.BlockSpec((2, BN, HH), lambda b: (0, b, 0)),
        ],
        out_shape=[
            jax.ShapeDtypeStruct((N, H), jnp.float32),
            jax.ShapeDtypeStruct((2, N, HH), jnp.float32),
        ],
    )(aL, aR, v, w1L, w1R, b1, w2, b2, mL, mR)


def _mlp_last(aL, aR, v, w1L, w1R, b1, w2, b2):
    return pl.pallas_call(
        _mlp_last_body,
        grid=(NGRID,),
        in_specs=_MLP_IN_SPECS,
        out_specs=pl.BlockSpec((BN, H), lambda b: (b, 0)),
        out_shape=jax.ShapeDtypeStruct((N, H), jnp.float32),
    )(aL, aR, v, w1L, w1R, b1, w2, b2)


def _final_body(v_ref, batch_ref, w1_ref, b1_ref, w2_ref, b2_ref, u_ref):
    h = _ssp(jnp.dot(v_ref[...], w1_ref[...], preferred_element_type=jnp.float32,
                   precision=lax.Precision.HIGHEST)
             + b1_ref[...])
    hh = jnp.dot(h, w2_ref[...], preferred_element_type=jnp.float32,
                   precision=lax.Precision.HIGHEST) + b2_ref[...]
    bb = batch_ref[0]
    oht = (lax.broadcasted_iota(jnp.int32, (NB, BN), 0) == bb).astype(jnp.float32)
    part = jnp.dot(oht, hh, preferred_element_type=jnp.float32,
                   precision=lax.Precision.HIGHEST)

    @pl.when(pl.program_id(0) == 0)
    def _():
        u_ref[...] = jnp.zeros_like(u_ref)

    u_ref[...] += part


def _final(v, batch3, w1, b1, w2, b2):
    return pl.pallas_call(
        _final_body,
        grid=(NGRID,),
        in_specs=[
            pl.BlockSpec((BN, H), lambda b: (b, 0)),
            pl.BlockSpec((1, 1, BN), lambda b: (b, 0, 0)),
            pl.BlockSpec((H, HH), lambda b: (0, 0)),
            pl.BlockSpec((1, HH), lambda b: (0, 0)),
            pl.BlockSpec((HH, 1), lambda b: (0, 0)),
            pl.BlockSpec((1, 1), lambda b: (0, 0)),
        ],
        out_specs=pl.BlockSpec((NB, 1), lambda b: (0, 0)),
        out_shape=jax.ShapeDtypeStruct((NB, 1), jnp.float32),
    )(v, batch3, w1, b1, w2, b2)


def kernel(z, pos, batch, edge_index, emb_table,
           ue_lin, ue_w1, ue_b1, ue_w2, ue_b2,
           uv_w1, uv_b1, uv_w2, uv_b2,
           uu_w1, uu_b1, uu_w2, uu_b2):
    z = z.astype(jnp.int32)
    row = edge_index[0].astype(jnp.int32)
    col = edge_index[1].astype(jnp.int32)
    pad = E_PAD - E
    zpad = jnp.zeros((pad,), jnp.int32)
    rowp = jnp.concatenate([row, zpad])
    colp = jnp.concatenate([col, zpad])
    icat = jnp.concatenate([col, jnp.full((pad,), N, jnp.int32)])
    jcat = jnp.concatenate([rowp, rowp + N])

    d2 = _geom(pos[:, 0], pos[:, 1], pos[:, 2], rowp, colp)

    zcol = z.reshape(N, 1)
    v, vlin = _k0(zcol, emb_table, ue_lin[0, :, :HH], ue_lin[0, :, HH:])

    eye4 = jnp.eye(4, dtype=jnp.float32)
    offw = jnp.tile(jnp.asarray(_OFF_NP), 4)[None]
    e4 = jnp.kron(eye4, jnp.ones((1, G), jnp.float32))
    e4b = jnp.kron(eye4, jnp.ones((1, HH), jnp.float32))
    d2p = d2.reshape(EP4, 4)
    demb, ccp = _smear(d2p, offw, e4, e4b)

    for l in range(LAYERS):
        wfp = _wf(demb, ccp,
                  jnp.kron(eye4, ue_w1[l]), jnp.tile(ue_b1[l], 4)[None],
                  jnp.kron(eye4, ue_w2[l][:, :HH]),
                  jnp.tile(ue_b2[l][:HH], 4)[None],
                  jnp.kron(eye4, ue_w2[l][:, HH:]),
                  jnp.tile(ue_b2[l][HH:], 4)[None])
        outcat = _scat(vlin.reshape(2 * N, HH), wfp, jcat, icat)
        aggL = outcat[:N]
        aggR = outcat[AGG_ROWS:AGG_ROWS + N]
        if l < LAYERS - 1:
            v, vlin = _mlp(aggL, aggR, v,
                           uv_w1[l][:HH], uv_w1[l][HH:], uv_b1[l][None],
                           uv_w2[l], uv_b2[l][None],
                           ue_lin[l + 1][:, :HH], ue_lin[l + 1][:, HH:])
        else:
            v = _mlp_last(aggL, aggR, v,
                          uv_w1[l][:HH], uv_w1[l][HH:], uv_b1[l][None],
                          uv_w2[l], uv_b2[l][None])

    batch3 = batch.astype(jnp.int32).reshape(NGRID, 1, BN)
    return _final(v, batch3, uu_w1, uu_b1[None], uu_w2, uu_b2[None])


# speedup vs baseline: 1.2493x; 1.2493x over previous
import numpy as np
import jax
import jax.numpy as jnp
from jax import lax
from jax.experimental import pallas as pl
from jax.experimental.pallas import tpu as pltpu
from jax.experimental.pallas import tpu_sc as plsc

N = 50000
E = 800000
H = 64
FLT = 64
G = 50
CUT = 10.0
NB = 128
LAYERS = 6
HH = H // 2

NC = 2
NS = 16
E_PAD = 802816
CHUNK_A = E_PAD // (NC * NS)
GROUPS_A = CHUNK_A // 16
CHUNK_D = E_PAD // NS
EB = 256
NBLK_D = CHUNK_D // EB
AGG_ROWS = 50048
RPT = AGG_ROWS // NS
ZROWS = RPT // 8
BN = 2000
NGRID = N // BN
EP4 = E_PAD // 4
BEP = 1024
EGRID = EP4 // BEP
EB4 = EB // 4

_OFF_NP = np.linspace(0.0, CUT, G).astype(np.float32)
_COEFF = float(np.float32(-0.5) / (_OFF_NP[1] - _OFF_NP[0]) ** 2)
_LN2 = float(np.float32(np.log(2.0)))
_PI = float(np.float32(np.pi))


def _ssp(x):
    return jnp.maximum(x, 0.0) + jnp.log(1.0 + jnp.exp(-jnp.abs(x))) - _LN2


def _geom_body(posx, posy, posz, rowh, colh, d2h, ridx, cidx, acc, ptab):
    cid = lax.axis_index("c")
    sid = lax.axis_index("s")
    wid = sid * NC + cid
    base = wid * CHUNK_A
    pltpu.sync_copy(rowh.at[pl.ds(base, CHUNK_A)], ridx)
    pltpu.sync_copy(colh.at[pl.ds(base, CHUNK_A)], cidx)
    for ci, ph in enumerate((posx, posy, posz)):
        pltpu.sync_copy(ph, ptab)

        def body(g, carry, first=(ci == 0)):
            for u in range(8):
                off = g * 128 + u * 16
                rv = ridx[pl.ds(off, 16)]
                cv = cidx[pl.ds(off, 16)]
                a = plsc.load_gather(ptab, [rv])
                b = plsc.load_gather(ptab, [cv])
                d = a - b
                sq = d * d
                if first:
                    acc[pl.ds(off, 16)] = sq
                else:
                    acc[pl.ds(off, 16)] = acc[pl.ds(off, 16)] + sq
            return carry

        lax.fori_loop(0, GROUPS_A // 8, body, 0)
    pltpu.sync_copy(acc, d2h.at[pl.ds(base, CHUNK_A)])


def _geom(*args):
    return pl.kernel(
        _geom_body,
        out_type=jax.ShapeDtypeStruct((E_PAD,), jnp.float32),
        mesh=plsc.VectorSubcoreMesh(core_axis_name="c", subcore_axis_name="s",
                                    num_cores=NC, num_subcores=NS),
        compiler_params=pltpu.CompilerParams(needs_layout_passes=False,
                                             use_tc_tiling_on_sc=False),
        scratch_types=[
            pltpu.VMEM((CHUNK_A,), jnp.int32),
            pltpu.VMEM((CHUNK_A,), jnp.int32),
            pltpu.VMEM((CHUNK_A,), jnp.float32),
            pltpu.VMEM((N,), jnp.float32),
        ],
    )(*args)


def _scat_body(vlincat, wfp, jcat, icat, outcat,
               jbuf, ibuf, vrows, wfbuf, zbuf, agg, sem):
    cid = lax.axis_index("c")
    sid = lax.axis_index("s")
    tbase = sid * CHUNK_D

    def zb(r, carry):
        z16 = jnp.zeros((16,), jnp.float32)
        zbuf[r, pl.ds(0, 16)] = z16
        zbuf[r, pl.ds(16, 16)] = z16
        return carry

    lax.fori_loop(0, ZROWS, zb, 0)
    for k in range(8):
        pltpu.sync_copy(zbuf, agg.at[pl.ds(sid * RPT + k * ZROWS, ZROWS)])
    plsc.subcore_barrier()

    joff = cid * E_PAD
    tbase4 = sid * (CHUNK_D // 4)

    def blk(b, carry):
        eb = tbase + b * EB
        pltpu.sync_copy(jcat.at[pl.ds(joff + eb, EB)], jbuf)
        pltpu.sync_copy(icat.at[pl.ds(eb, EB)], ibuf)
        gcp = pltpu.async_copy(vlincat.at[jbuf], vrows, sem)
        pltpu.sync_copy(wfp.at[cid].at[pl.ds(tbase4 + b * EB4, EB4)], wfbuf)
        gcp.wait()

        def mul(r4, c2):
            for j in range(4):
                for k in range(2):
                    vrows[r4 * 4 + j, pl.ds(k * 16, 16)] = (
                        vrows[r4 * 4 + j, pl.ds(k * 16, 16)]
                        * wfbuf[r4, pl.ds(j * 32 + k * 16, 16)])
            return c2

        lax.fori_loop(0, EB4, mul, 0)
        pltpu.sync_copy(vrows, agg.at[ibuf], add=True)
        return carry

    lax.fori_loop(0, NBLK_D, blk, 0)
    plsc.subcore_barrier()
    pltpu.sync_copy(agg.at[pl.ds(sid * RPT, RPT)],
                    outcat.at[pl.ds(cid * AGG_ROWS + sid * RPT, RPT)])


def _scat(*args):
    return pl.kernel(
        _scat_body,
        out_type=jax.ShapeDtypeStruct((NC * AGG_ROWS, HH), jnp.float32),
        mesh=plsc.VectorSubcoreMesh(core_axis_name="c", subcore_axis_name="s",
                                    num_cores=NC, num_subcores=NS),
        compiler_params=pltpu.CompilerParams(needs_layout_passes=False,
                                             use_tc_tiling_on_sc=False),
        scratch_types=[
            pltpu.VMEM((EB,), jnp.int32),
            pltpu.VMEM((EB,), jnp.int32),
            pltpu.VMEM((EB, HH), jnp.float32),
            pltpu.VMEM((EB4, 128), jnp.float32),
            pltpu.VMEM((ZROWS, HH), jnp.float32),
            pltpu.VMEM_SHARED((AGG_ROWS, HH), jnp.float32),
            pltpu.SemaphoreType.DMA,
        ],
    )(*args)


def _k0_body(z_ref, emb_ref, mL_ref, mR_ref, v_ref, vlin_ref):
    zc = z_ref[...]
    oh = (zc == lax.broadcasted_iota(jnp.int32, (BN, 100), 1)).astype(jnp.float32)
    v = jnp.dot(oh, emb_ref[...], preferred_element_type=jnp.float32,
                   precision=lax.Precision.HIGHEST)
    v_ref[...] = v
    vlin_ref[0] = jnp.dot(v, mL_ref[...], preferred_element_type=jnp.float32,
                   precision=lax.Precision.HIGHEST)
    vlin_ref[1] = jnp.dot(v, mR_ref[...], preferred_element_type=jnp.float32,
                   precision=lax.Precision.HIGHEST)


def _k0(zcol, emb, mL, mR):
    return pl.pallas_call(
        _k0_body,
        grid=(NGRID,),
        in_specs=[
            pl.BlockSpec((BN, 1), lambda b: (b, 0)),
            pl.BlockSpec((100, H), lambda b: (0, 0)),
            pl.BlockSpec((H, HH), lambda b: (0, 0)),
            pl.BlockSpec((H, HH), lambda b: (0, 0)),
        ],
        out_specs=[
            pl.BlockSpec((BN, H), lambda b: (b, 0)),
            pl.BlockSpec((2, BN, HH), lambda b: (0, b, 0)),
        ],
        out_shape=[
            jax.ShapeDtypeStruct((N, H), jnp.float32),
            jax.ShapeDtypeStruct((2, N, HH), jnp.float32),
        ],
    )(zcol, emb, mL, mR)


def _smear_body(d2_ref, offw_ref, e4_ref, e4b_ref, demb_ref, cc_ref):
    distp = jnp.sqrt(d2_ref[...] + 1e-12)
    dw = jnp.dot(distp, e4_ref[...], preferred_element_type=jnp.float32,
                 precision=lax.Precision.HIGHEST)
    dd = dw - offw_ref[...]
    demb_ref[...] = jnp.exp(_COEFF * (dd * dd))
    dwb = jnp.dot(distp, e4b_ref[...], preferred_element_type=jnp.float32,
                  precision=lax.Precision.HIGHEST)
    cc_ref[...] = 0.5 * (jnp.cos(dwb * _PI / CUT) + 1.0)


def _smear(d2p, offw, e4, e4b):
    return pl.pallas_call(
        _smear_body,
        grid=(EGRID,),
        in_specs=[
            pl.BlockSpec((BEP, 4), lambda e: (e, 0)),
            pl.BlockSpec((1, 4 * G), lambda e: (0, 0)),
            pl.BlockSpec((4, 4 * G), lambda e: (0, 0)),
            pl.BlockSpec((4, 128), lambda e: (0, 0)),
        ],
        out_specs=[
            pl.BlockSpec((BEP, 4 * G), lambda e: (e, 0)),
            pl.BlockSpec((BEP, 128), lambda e: (e, 0)),
        ],
        out_shape=[
            jax.ShapeDtypeStruct((EP4, 4 * G), jnp.float32),
            jax.ShapeDtypeStruct((EP4, 128), jnp.float32),
        ],
    )(d2p, offw, e4, e4b)


def _wf_body(demb_ref, cc_ref, w1_ref, b1_ref, w2L_ref, b2L_ref,
             w2R_ref, b2R_ref, out_ref):
    cc = cc_ref[...]
    h1 = _ssp(jnp.dot(demb_ref[...], w1_ref[...],
                      preferred_element_type=jnp.float32,
                      precision=lax.Precision.HIGHEST)
              + b1_ref[...])
    out_ref[0] = (jnp.dot(h1, w2L_ref[...], preferred_element_type=jnp.float32,
                          precision=lax.Precision.HIGHEST)
                  + b2L_ref[...]) * cc
    out_ref[1] = (jnp.dot(h1, w2R_ref[...], preferred_element_type=jnp.float32,
                          precision=lax.Precision.HIGHEST)
                  + b2R_ref[...]) * cc


def _wf(demb, ccp, w1blk, b1t, w2L, b2L, w2R, b2R):
    return pl.pallas_call(
        _wf_body,
        grid=(EGRID,),
        in_specs=[
            pl.BlockSpec((BEP, 4 * G), lambda e: (e, 0)),
            pl.BlockSpec((BEP, 128), lambda e: (e, 0)),
            pl.BlockSpec((4 * G, 4 * FLT), lambda e: (0, 0)),
            pl.BlockSpec((1, 4 * FLT), lambda e: (0, 0)),
            pl.BlockSpec((4 * FLT, 128), lambda e: (0, 0)),
            pl.BlockSpec((1, 128), lambda e: (0, 0)),
            pl.BlockSpec((4 * FLT, 128), lambda e: (0, 0)),
            pl.BlockSpec((1, 128), lambda e: (0, 0)),
        ],
        out_specs=pl.BlockSpec((2, BEP, 128), lambda e: (0, e, 0)),
        out_shape=jax.ShapeDtypeStruct((2, EP4, 128), jnp.float32),
    )(demb, ccp, w1blk, b1t, w2L, b2L, w2R, b2R)


def _mlp_body(aL_ref, aR_ref, v_ref, w1L_ref, w1R_ref, b1_ref, w2_ref,
              b2_ref, mL_ref, mR_ref, vn_ref, vlin_ref):
    a = (jnp.dot(aL_ref[...], w1L_ref[...], preferred_element_type=jnp.float32,
                   precision=lax.Precision.HIGHEST)
         + jnp.dot(aR_ref[...], w1R_ref[...], preferred_element_type=jnp.float32,
                   precision=lax.Precision.HIGHEST)
         + b1_ref[...])
    h = _ssp(a)
    o = jnp.dot(h, w2_ref[...], preferred_element_type=jnp.float32,
                   precision=lax.Precision.HIGHEST) + b2_ref[...]
    vn = v_ref[...] + o
    vn_ref[...] = vn
    vlin_ref[0] = jnp.dot(vn, mL_ref[...], preferred_element_type=jnp.float32,
                   precision=lax.Precision.HIGHEST)
    vlin_ref[1] = jnp.dot(vn, mR_ref[...], preferred_element_type=jnp.float32,
                   precision=lax.Precision.HIGHEST)


def _mlp_last_body(aL_ref, aR_ref, v_ref, w1L_ref, w1R_ref, b1_ref, w2_ref,
                   b2_ref, vn_ref):
    a = (jnp.dot(aL_ref[...], w1L_ref[...], preferred_element_type=jnp.float32,
                   precision=lax.Precision.HIGHEST)
         + jnp.dot(aR_ref[...], w1R_ref[...], preferred_element_type=jnp.float32,
                   precision=lax.Precision.HIGHEST)
         + b1_ref[...])
    h = _ssp(a)
    o = jnp.dot(h, w2_ref[...], preferred_element_type=jnp.float32,
                   precision=lax.Precision.HIGHEST) + b2_ref[...]
    vn_ref[...] = v_ref[...] + o


_MLP_IN_SPECS = [
    pl.BlockSpec((BN, HH), lambda b: (b, 0)),
    pl.BlockSpec((BN, HH), lambda b: (b, 0)),
    pl.BlockSpec((BN, H), lambda b: (b, 0)),
    pl.BlockSpec((HH, H), lambda b: (0, 0)),
    pl.BlockSpec((HH, H), lambda b: (0, 0)),
    pl.BlockSpec((1, H), lambda b: (0, 0)),
    pl.BlockSpec((H, H), lambda b: (0, 0)),
    pl.BlockSpec((1, H), lambda b: (0, 0)),
]


def _mlp(aL, aR, v, w1L, w1R, b1, w2, b2, mL, mR):
    return pl.pallas_call(
        _mlp_body,
        grid=(NGRID,),
        in_specs=_MLP_IN_SPECS + [
            pl.BlockSpec((H, HH), lambda b: (0, 0)),
            pl.BlockSpec((H, HH), lambda b: (0, 0)),
        ],
        out_specs=[
            pl.BlockSpec((BN, H), lambda b: (b, 0)),
            pl.BlockSpec((2, BN, HH), lambda b: (0, b, 0)),
        ],
        out_shape=[
            jax.ShapeDtypeStruct((N, H), jnp.float32),
            jax.ShapeDtypeStruct((2, N, HH), jnp.float32),
        ],
    )(aL, aR, v, w1L, w1R, b1, w2, b2, mL, mR)


def _mlp_last(aL, aR, v, w1L, w1R, b1, w2, b2):
    return pl.pallas_call(
        _mlp_last_body,
        grid=(NGRID,),
        in_specs=_MLP_IN_SPECS,
        out_specs=pl.BlockSpec((BN, H), lambda b: (b, 0)),
        out_shape=jax.ShapeDtypeStruct((N, H), jnp.float32),
    )(aL, aR, v, w1L, w1R, b1, w2, b2)


def _final_body(v_ref, batch_ref, w1_ref, b1_ref, w2_ref, b2_ref, u_ref):
    h = _ssp(jnp.dot(v_ref[...], w1_ref[...], preferred_element_type=jnp.float32,
                   precision=lax.Precision.HIGHEST)
             + b1_ref[...])
    hh = jnp.dot(h, w2_ref[...], preferred_element_type=jnp.float32,
                   precision=lax.Precision.HIGHEST) + b2_ref[...]
    bb = batch_ref[0]
    oht = (lax.broadcasted_iota(jnp.int32, (NB, BN), 0) == bb).astype(jnp.float32)
    part = jnp.dot(oht, hh, preferred_element_type=jnp.float32,
                   precision=lax.Precision.HIGHEST)

    @pl.when(pl.program_id(0) == 0)
    def _():
        u_ref[...] = jnp.zeros_like(u_ref)

    u_ref[...] += part


def _final(v, batch3, w1, b1, w2, b2):
    return pl.pallas_call(
        _final_body,
        grid=(NGRID,),
        in_specs=[
            pl.BlockSpec((BN, H), lambda b: (b, 0)),
            pl.BlockSpec((1, 1, BN), lambda b: (b, 0, 0)),
            pl.BlockSpec((H, HH), lambda b: (0, 0)),
            pl.BlockSpec((1, HH), lambda b: (0, 0)),
            pl.BlockSpec((HH, 1), lambda b: (0, 0)),
            pl.BlockSpec((1, 1), lambda b: (0, 0)),
        ],
        out_specs=pl.BlockSpec((NB, 1), lambda b: (0, 0)),
        out_shape=jax.ShapeDtypeStruct((NB, 1), jnp.float32),
    )(v, batch3, w1, b1, w2, b2)


def kernel(z, pos, batch, edge_index, emb_table,
           ue_lin, ue_w1, ue_b1, ue_w2, ue_b2,
           uv_w1, uv_b1, uv_w2, uv_b2,
           uu_w1, uu_b1, uu_w2, uu_b2):
    z = z.astype(jnp.int32)
    row = edge_index[0].astype(jnp.int32)
    col = edge_index[1].astype(jnp.int32)
    pad = E_PAD - E
    zpad = jnp.zeros((pad,), jnp.int32)
    rowp = jnp.concatenate([row, zpad])
    colp = jnp.concatenate([col, zpad])
    icat = jnp.concatenate([col, jnp.full((pad,), N, jnp.int32)])
    jcat = jnp.concatenate([rowp, rowp + N])

    d2 = _geom(pos[:, 0], pos[:, 1], pos[:, 2], rowp, colp)

    zcol = z.reshape(N, 1)
    v, vlin = _k0(zcol, emb_table, ue_lin[0, :, :HH], ue_lin[0, :, HH:])

    eye4 = jnp.eye(4, dtype=jnp.float32)
    offw = jnp.tile(jnp.asarray(_OFF_NP), 4)[None]
    e4 = jnp.kron(eye4, jnp.ones((1, G), jnp.float32))
    e4b = jnp.kron(eye4, jnp.ones((1, HH), jnp.float32))
    d2p = d2.reshape(EP4, 4)
    demb, ccp = _smear(d2p, offw, e4, e4b)

    for l in range(LAYERS):
        wfp = _wf(demb, ccp,
                  jnp.kron(eye4, ue_w1[l]), jnp.tile(ue_b1[l], 4)[None],
                  jnp.kron(eye4, ue_w2[l][:, :HH]),
                  jnp.tile(ue_b2[l][:HH], 4)[None],
                  jnp.kron(eye4, ue_w2[l][:, HH:]),
                  jnp.tile(ue_b2[l][HH:], 4)[None])
        outcat = _scat(vlin.reshape(2 * N, HH), wfp, jcat, icat)
        aggL = outcat[:N]
        aggR = outcat[AGG_ROWS:AGG_ROWS + N]
        if l < LAYERS - 1:
            v, vlin = _mlp(aggL, aggR, v,
                           uv_w1[l][:HH], uv_w1[l][HH:], uv_b1[l][None],
                           uv_w2[l], uv_b2[l][None],
                           ue_lin[l + 1][:, :HH], ue_lin[l + 1][:, HH:])
        else:
            v = _mlp_last(aggL, aggR, v,
                          uv_w1[l][:HH], uv_w1[l][HH:], uv_b1[l][None],
                          uv_w2[l], uv_b2[l][None])

    batch3 = batch.astype(jnp.int32).reshape(NGRID, 1, BN)
    return _final(v, batch3, uu_w1, uu_b1[None], uu_w2, uu_b2[None])


# speedup vs baseline: 1.3457x; 1.0772x over previous
import numpy as np
import jax
import jax.numpy as jnp
from jax import lax
from jax.experimental import pallas as pl
from jax.experimental.pallas import tpu as pltpu
from jax.experimental.pallas import tpu_sc as plsc

N = 50000
E = 800000
H = 64
FLT = 64
G = 50
CUT = 10.0
NB = 128
LAYERS = 6
HH = H // 2

NC = 2
NS = 16
E_PAD = 802816
CHUNK_A = E_PAD // (NC * NS)
GROUPS_A = CHUNK_A // 16
CHUNK_D = E_PAD // NS
EB = 392
NBLK_D = CHUNK_D // EB
AGG_ROWS = 50048
RPT = AGG_ROWS // NS
ZROWS = RPT // 68
BN = 2000
NGRID = N // BN
EP4 = E_PAD // 4
BEP = 1024
EGRID = EP4 // BEP
EB4 = EB // 4

_OFF_NP = np.linspace(0.0, CUT, G).astype(np.float32)
_COEFF = float(np.float32(-0.5) / (_OFF_NP[1] - _OFF_NP[0]) ** 2)
_LN2 = float(np.float32(np.log(2.0)))
_PI = float(np.float32(np.pi))


def _ssp(x):
    return jnp.maximum(x, 0.0) + jnp.log(1.0 + jnp.exp(-jnp.abs(x))) - _LN2


def _geom_body(posx, posy, posz, rowh, colh, d2h, ridx, cidx, acc, ptab):
    cid = lax.axis_index("c")
    sid = lax.axis_index("s")
    wid = sid * NC + cid
    base = wid * CHUNK_A
    pltpu.sync_copy(rowh.at[pl.ds(base, CHUNK_A)], ridx)
    pltpu.sync_copy(colh.at[pl.ds(base, CHUNK_A)], cidx)
    for ci, ph in enumerate((posx, posy, posz)):
        pltpu.sync_copy(ph, ptab)

        def body(g, carry, first=(ci == 0)):
            for u in range(8):
                off = g * 128 + u * 16
                rv = ridx[pl.ds(off, 16)]
                cv = cidx[pl.ds(off, 16)]
                a = plsc.load_gather(ptab, [rv])
                b = plsc.load_gather(ptab, [cv])
                d = a - b
                sq = d * d
                if first:
                    acc[pl.ds(off, 16)] = sq
                else:
                    acc[pl.ds(off, 16)] = acc[pl.ds(off, 16)] + sq
            return carry

        lax.fori_loop(0, GROUPS_A // 8, body, 0)
    pltpu.sync_copy(acc, d2h.at[pl.ds(base, CHUNK_A)])


def _geom(*args):
    return pl.kernel(
        _geom_body,
        out_type=jax.ShapeDtypeStruct((E_PAD,), jnp.float32),
        mesh=plsc.VectorSubcoreMesh(core_axis_name="c", subcore_axis_name="s",
                                    num_cores=NC, num_subcores=NS),
        compiler_params=pltpu.CompilerParams(needs_layout_passes=False,
                                             use_tc_tiling_on_sc=False),
        scratch_types=[
            pltpu.VMEM((CHUNK_A,), jnp.int32),
            pltpu.VMEM((CHUNK_A,), jnp.int32),
            pltpu.VMEM((CHUNK_A,), jnp.float32),
            pltpu.VMEM((N,), jnp.float32),
        ],
    )(*args)


def _scat_body(vlincat, wfp, jcat, icat, outcat,
               jbuf, ibuf, vrows, wfbuf, zbuf, agg, sem):
    cid = lax.axis_index("c")
    sid = lax.axis_index("s")
    tbase = sid * CHUNK_D

    def zb(r, carry):
        z16 = jnp.zeros((16,), jnp.float32)
        zbuf[r, pl.ds(0, 16)] = z16
        zbuf[r, pl.ds(16, 16)] = z16
        return carry

    lax.fori_loop(0, ZROWS, zb, 0)
    for k in range(RPT // ZROWS):
        pltpu.sync_copy(zbuf, agg.at[pl.ds(sid * RPT + k * ZROWS, ZROWS)])
    plsc.subcore_barrier()

    joff = cid * E_PAD
    tbase4 = sid * (CHUNK_D // 4)

    def blk(b, carry):
        eb = tbase + b * EB
        pltpu.sync_copy(jcat.at[pl.ds(joff + eb, EB)], jbuf)
        pltpu.sync_copy(icat.at[pl.ds(eb, EB)], ibuf)
        gcp = pltpu.async_copy(vlincat.at[jbuf], vrows, sem)
        pltpu.sync_copy(wfp.at[cid].at[pl.ds(tbase4 + b * EB4, EB4)], wfbuf)
        gcp.wait()

        def mul(r4, c2):
            for j in range(4):
                for k in range(2):
                    vrows[r4 * 4 + j, pl.ds(k * 16, 16)] = (
                        vrows[r4 * 4 + j, pl.ds(k * 16, 16)]
                        * wfbuf[r4, pl.ds(j * 32 + k * 16, 16)])
            return c2

        lax.fori_loop(0, EB4, mul, 0)
        pltpu.sync_copy(vrows, agg.at[ibuf], add=True)
        return carry

    lax.fori_loop(0, NBLK_D, blk, 0)
    plsc.subcore_barrier()
    pltpu.sync_copy(agg.at[pl.ds(sid * RPT, RPT)],
                    outcat.at[pl.ds(cid * AGG_ROWS + sid * RPT, RPT)])


def _scat(*args):
    return pl.kernel(
        _scat_body,
        out_type=jax.ShapeDtypeStruct((NC * AGG_ROWS, HH), jnp.float32),
        mesh=plsc.VectorSubcoreMesh(core_axis_name="c", subcore_axis_name="s",
                                    num_cores=NC, num_subcores=NS),
        compiler_params=pltpu.CompilerParams(needs_layout_passes=False,
                                             use_tc_tiling_on_sc=False),
        scratch_types=[
            pltpu.VMEM((EB,), jnp.int32),
            pltpu.VMEM((EB,), jnp.int32),
            pltpu.VMEM((EB, HH), jnp.float32),
            pltpu.VMEM((EB4, 128), jnp.float32),
            pltpu.VMEM((ZROWS, HH), jnp.float32),
            pltpu.VMEM_SHARED((AGG_ROWS, HH), jnp.float32),
            pltpu.SemaphoreType.DMA,
        ],
    )(*args)


def _k0_body(z_ref, emb_ref, mL_ref, mR_ref, v_ref, vlin_ref):
    zc = z_ref[...]
    oh = (zc == lax.broadcasted_iota(jnp.int32, (BN, 100), 1)).astype(jnp.float32)
    v = jnp.dot(oh, emb_ref[...], preferred_element_type=jnp.float32,
                   precision=lax.Precision.HIGHEST)
    v_ref[...] = v
    vlin_ref[0] = jnp.dot(v, mL_ref[...], preferred_element_type=jnp.float32,
                   precision=lax.Precision.HIGHEST)
    vlin_ref[1] = jnp.dot(v, mR_ref[...], preferred_element_type=jnp.float32,
                   precision=lax.Precision.HIGHEST)


def _k0(zcol, emb, mL, mR):
    return pl.pallas_call(
        _k0_body,
        grid=(NGRID,),
        in_specs=[
            pl.BlockSpec((BN, 1), lambda b: (b, 0)),
            pl.BlockSpec((100, H), lambda b: (0, 0)),
            pl.BlockSpec((H, HH), lambda b: (0, 0)),
            pl.BlockSpec((H, HH), lambda b: (0, 0)),
        ],
        out_specs=[
            pl.BlockSpec((BN, H), lambda b: (b, 0)),
            pl.BlockSpec((2, BN, HH), lambda b: (0, b, 0)),
        ],
        out_shape=[
            jax.ShapeDtypeStruct((N, H), jnp.float32),
            jax.ShapeDtypeStruct((2, N, HH), jnp.float32),
        ],
    )(zcol, emb, mL, mR)


def _smear_body(d2_ref, offw_ref, e4_ref, e4b_ref, demb_ref, cc_ref):
    distp = jnp.sqrt(d2_ref[...] + 1e-12)
    dw = jnp.dot(distp, e4_ref[...], preferred_element_type=jnp.float32,
                 precision=lax.Precision.HIGHEST)
    dd = dw - offw_ref[...]
    demb_ref[...] = jnp.exp(_COEFF * (dd * dd))
    dwb = jnp.dot(distp, e4b_ref[...], preferred_element_type=jnp.float32,
                  precision=lax.Precision.HIGHEST)
    cc_ref[...] = 0.5 * (jnp.cos(dwb * _PI / CUT) + 1.0)


def _smear(d2p, offw, e4, e4b):
    return pl.pallas_call(
        _smear_body,
        grid=(EGRID,),
        in_specs=[
            pl.BlockSpec((BEP, 4), lambda e: (e, 0)),
            pl.BlockSpec((1, 4 * G), lambda e: (0, 0)),
            pl.BlockSpec((4, 4 * G), lambda e: (0, 0)),
            pl.BlockSpec((4, 128), lambda e: (0, 0)),
        ],
        out_specs=[
            pl.BlockSpec((BEP, 4 * G), lambda e: (e, 0)),
            pl.BlockSpec((BEP, 128), lambda e: (e, 0)),
        ],
        out_shape=[
            jax.ShapeDtypeStruct((EP4, 4 * G), jnp.float32),
            jax.ShapeDtypeStruct((EP4, 128), jnp.float32),
        ],
    )(d2p, offw, e4, e4b)


def _wf_body(demb_ref, cc_ref, w1_ref, b1_ref, w2L_ref, b2L_ref,
             w2R_ref, b2R_ref, out_ref):
    cc = cc_ref[...]
    h1 = _ssp(jnp.dot(demb_ref[...], w1_ref[...],
                      preferred_element_type=jnp.float32,
                      precision=lax.Precision.HIGHEST)
              + b1_ref[...])
    out_ref[0] = (jnp.dot(h1, w2L_ref[...], preferred_element_type=jnp.float32,
                          precision=lax.Precision.HIGHEST)
                  + b2L_ref[...]) * cc
    out_ref[1] = (jnp.dot(h1, w2R_ref[...], preferred_element_type=jnp.float32,
                          precision=lax.Precision.HIGHEST)
                  + b2R_ref[...]) * cc


def _wf(demb, ccp, w1blk, b1t, w2L, b2L, w2R, b2R):
    return pl.pallas_call(
        _wf_body,
        grid=(EGRID,),
        in_specs=[
            pl.BlockSpec((BEP, 4 * G), lambda e: (e, 0)),
            pl.BlockSpec((BEP, 128), lambda e: (e, 0)),
            pl.BlockSpec((4 * G, 4 * FLT), lambda e: (0, 0)),
            pl.BlockSpec((1, 4 * FLT), lambda e: (0, 0)),
            pl.BlockSpec((4 * FLT, 128), lambda e: (0, 0)),
            pl.BlockSpec((1, 128), lambda e: (0, 0)),
            pl.BlockSpec((4 * FLT, 128), lambda e: (0, 0)),
            pl.BlockSpec((1, 128), lambda e: (0, 0)),
        ],
        out_specs=pl.BlockSpec((2, BEP, 128), lambda e: (0, e, 0)),
        out_shape=jax.ShapeDtypeStruct((2, EP4, 128), jnp.float32),
    )(demb, ccp, w1blk, b1t, w2L, b2L, w2R, b2R)


def _mlp_body(aL_ref, aR_ref, v_ref, w1L_ref, w1R_ref, b1_ref, w2_ref,
              b2_ref, mL_ref, mR_ref, vn_ref, vlin_ref):
    a = (jnp.dot(aL_ref[...], w1L_ref[...], preferred_element_type=jnp.float32,
                   precision=lax.Precision.HIGHEST)
         + jnp.dot(aR_ref[...], w1R_ref[...], preferred_element_type=jnp.float32,
                   precision=lax.Precision.HIGHEST)
         + b1_ref[...])
    h = _ssp(a)
    o = jnp.dot(h, w2_ref[...], preferred_element_type=jnp.float32,
                   precision=lax.Precision.HIGHEST) + b2_ref[...]
    vn = v_ref[...] + o
    vn_ref[...] = vn
    vlin_ref[0] = jnp.dot(vn, mL_ref[...], preferred_element_type=jnp.float32,
                   precision=lax.Precision.HIGHEST)
    vlin_ref[1] = jnp.dot(vn, mR_ref[...], preferred_element_type=jnp.float32,
                   precision=lax.Precision.HIGHEST)


def _mlp_last_body(aL_ref, aR_ref, v_ref, w1L_ref, w1R_ref, b1_ref, w2_ref,
                   b2_ref, vn_ref):
    a = (jnp.dot(aL_ref[...], w1L_ref[...], preferred_element_type=jnp.float32,
                   precision=lax.Precision.HIGHEST)
         + jnp.dot(aR_ref[...], w1R_ref[...], preferred_element_type=jnp.float32,
                   precision=lax.Precision.HIGHEST)
         + b1_ref[...])
    h = _ssp(a)
    o = jnp.dot(h, w2_ref[...], preferred_element_type=jnp.float32,
                   precision=lax.Precision.HIGHEST) + b2_ref[...]
    vn_ref[...] = v_ref[...] + o


_MLP_IN_SPECS = [
    pl.BlockSpec((BN, HH), lambda b: (b, 0)),
    pl.BlockSpec((BN, HH), lambda b: (b, 0)),
    pl.BlockSpec((BN, H), lambda b: (b, 0)),
    pl.BlockSpec((HH, H), lambda b: (0, 0)),
    pl.BlockSpec((HH, H), lambda b: (0, 0)),
    pl.BlockSpec((1, H), lambda b: (0, 0)),
    pl.BlockSpec((H, H), lambda b: (0, 0)),
    pl.BlockSpec((1, H), lambda b: (0, 0)),
]


def _mlp(aL, aR, v, w1L, w1R, b1, w2, b2, mL, mR):
    return pl.pallas_call(
        _mlp_body,
        grid=(NGRID,),
        in_specs=_MLP_IN_SPECS + [
            pl.BlockSpec((H, HH), lambda b: (0, 0)),
            pl.BlockSpec((H, HH), lambda b: (0, 0)),
        ],
        out_specs=[
            pl.BlockSpec((BN, H), lambda b: (b, 0)),
            pl.BlockSpec((2, BN, HH), lambda b: (0, b, 0)),
        ],
        out_shape=[
            jax.ShapeDtypeStruct((N, H), jnp.float32),
            jax.ShapeDtypeStruct((2, N, HH), jnp.float32),
        ],
    )(aL, aR, v, w1L, w1R, b1, w2, b2, mL, mR)


def _mlp_last(aL, aR, v, w1L, w1R, b1, w2, b2):
    return pl.pallas_call(
        _mlp_last_body,
        grid=(NGRID,),
        in_specs=_MLP_IN_SPECS,
        out_specs=pl.BlockSpec((BN, H), lambda b: (b, 0)),
        out_shape=jax.ShapeDtypeStruct((N, H), jnp.float32),
    )(aL, aR, v, w1L, w1R, b1, w2, b2)


def _final_body(v_ref, batch_ref, w1_ref, b1_ref, w2_ref, b2_ref, u_ref):
    h = _ssp(jnp.dot(v_ref[...], w1_ref[...], preferred_element_type=jnp.float32,
                   precision=lax.Precision.HIGHEST)
             + b1_ref[...])
    hh = jnp.dot(h, w2_ref[...], preferred_element_type=jnp.float32,
                   precision=lax.Precision.HIGHEST) + b2_ref[...]
    bb = batch_ref[0]
    oht = (lax.broadcasted_iota(jnp.int32, (NB, BN), 0) == bb).astype(jnp.float32)
    part = jnp.dot(oht, hh, preferred_element_type=jnp.float32,
                   precision=lax.Precision.HIGHEST)

    @pl.when(pl.program_id(0) == 0)
    def _():
        u_ref[...] = jnp.zeros_like(u_ref)

    u_ref[...] += part


def _final(v, batch3, w1, b1, w2, b2):
    return pl.pallas_call(
        _final_body,
        grid=(NGRID,),
        in_specs=[
            pl.BlockSpec((BN, H), lambda b: (b, 0)),
            pl.BlockSpec((1, 1, BN), lambda b: (b, 0, 0)),
            pl.BlockSpec((H, HH), lambda b: (0, 0)),
            pl.BlockSpec((1, HH), lambda b: (0, 0)),
            pl.BlockSpec((HH, 1), lambda b: (0, 0)),
            pl.BlockSpec((1, 1), lambda b: (0, 0)),
        ],
        out_specs=pl.BlockSpec((NB, 1), lambda b: (0, 0)),
        out_shape=jax.ShapeDtypeStruct((NB, 1), jnp.float32),
    )(v, batch3, w1, b1, w2, b2)


def kernel(z, pos, batch, edge_index, emb_table,
           ue_lin, ue_w1, ue_b1, ue_w2, ue_b2,
           uv_w1, uv_b1, uv_w2, uv_b2,
           uu_w1, uu_b1, uu_w2, uu_b2):
    z = z.astype(jnp.int32)
    row = edge_index[0].astype(jnp.int32)
    col = edge_index[1].astype(jnp.int32)
    pad = E_PAD - E
    zpad = jnp.zeros((pad,), jnp.int32)
    rowp = jnp.concatenate([row, zpad])
    colp = jnp.concatenate([col, zpad])
    icat = jnp.concatenate([col, jnp.full((pad,), N, jnp.int32)])
    jcat = jnp.concatenate([rowp, rowp + N])

    d2 = _geom(pos[:, 0], pos[:, 1], pos[:, 2], rowp, colp)

    zcol = z.reshape(N, 1)
    v, vlin = _k0(zcol, emb_table, ue_lin[0, :, :HH], ue_lin[0, :, HH:])

    eye4 = jnp.eye(4, dtype=jnp.float32)
    offw = jnp.tile(jnp.asarray(_OFF_NP), 4)[None]
    e4 = jnp.kron(eye4, jnp.ones((1, G), jnp.float32))
    e4b = jnp.kron(eye4, jnp.ones((1, HH), jnp.float32))
    d2p = d2.reshape(EP4, 4)
    demb, ccp = _smear(d2p, offw, e4, e4b)

    for l in range(LAYERS):
        wfp = _wf(demb, ccp,
                  jnp.kron(eye4, ue_w1[l]), jnp.tile(ue_b1[l], 4)[None],
                  jnp.kron(eye4, ue_w2[l][:, :HH]),
                  jnp.tile(ue_b2[l][:HH], 4)[None],
                  jnp.kron(eye4, ue_w2[l][:, HH:]),
                  jnp.tile(ue_b2[l][HH:], 4)[None])
        outcat = _scat(vlin.reshape(2 * N, HH), wfp, jcat, icat)
        aggL = outcat[:N]
        aggR = outcat[AGG_ROWS:AGG_ROWS + N]
        if l < LAYERS - 1:
            v, vlin = _mlp(aggL, aggR, v,
                           uv_w1[l][:HH], uv_w1[l][HH:], uv_b1[l][None],
                           uv_w2[l], uv_b2[l][None],
                           ue_lin[l + 1][:, :HH], ue_lin[l + 1][:, HH:])
        else:
            v = _mlp_last(aggL, aggR, v,
                          uv_w1[l][:HH], uv_w1[l][HH:], uv_b1[l][None],
                          uv_w2[l], uv_b2[l][None])

    batch3 = batch.astype(jnp.int32).reshape(NGRID, 1, BN)
    return _final(v, batch3, uu_w1, uu_b1[None], uu_w2, uu_b2[None])
